# baseline (speedup 1.0000x reference)
import math

import jax
import jax.numpy as jnp
from jax.experimental import pallas as pl
from jax.experimental.pallas import tpu as pltpu


def _body(x_ref, pos_ref, batch_ref, bcol_ref, xv_ref, wvsc_ref, wnsc_ref,
          wtp_ref, wn2v_ref, xvo_ref, out_ref, seg_ref, y2_ref):
    ph = pl.program_id(0)
    i = pl.program_id(1)
    d = x_ref.shape[1]
    g = seg_ref.shape[0]

    @pl.when(ph == 0)
    def _accumulate():
        @pl.when(i == 0)
        def _init():
            seg_ref[...] = jnp.zeros_like(seg_ref)

        x = x_ref[...]
        z = jnp.dot(x, wtp_ref[...], preferred_element_type=jnp.float32)
        pos = pos_ref[...]
        m = pos[:, 0:1] * z[:, 0:d]
        for j in range(1, pos.shape[1]):
            m = m + pos[:, j:j + 1] * z[:, j * d:(j + 1) * d]
        bb = batch_ref[0]
        onehot_t = (jax.lax.broadcasted_iota(jnp.int32, (g, bb.shape[1]), 0)
                    == bb).astype(jnp.float32)
        seg_ref[...] += jnp.dot(onehot_t, m, preferred_element_type=jnp.float32)

    @pl.when((ph == 1) & (i == 0))
    def _combine():
        sv = jnp.dot(xv_ref[...], wvsc_ref[...],
                     preferred_element_type=jnp.float32)
        mv = seg_ref[...]
        mv = mv * jax.nn.sigmoid(mv)
        xvo = (sv + mv) * (1.0 / math.sqrt(2.0))
        xvo_ref[...] = xvo
        y2_ref[...] = jnp.dot(xvo, wn2v_ref[...],
                              preferred_element_type=jnp.float32)

    @pl.when(ph == 1)
    def _node_out():
        x = x_ref[...]
        s = jnp.dot(x, wnsc_ref[...], preferred_element_type=jnp.float32)
        bcol = bcol_ref[...]
        onehot = (bcol == jax.lax.broadcasted_iota(
            jnp.int32, (x.shape[0], g), 1)).astype(jnp.float32)
        gath = jnp.dot(onehot, y2_ref[...], preferred_element_type=jnp.float32)
        out_ref[...] = (s + gath * jax.nn.sigmoid(gath)) * 0.5


def kernel(x_virtual, x_node, node_pos_sh, batch, W_vsc, W_nsc, W_tp, W_n2v):
    n, d = x_node.shape
    p = node_pos_sh.shape[1]
    g = x_virtual.shape[0]
    avg_nodes = n / g

    B = 2000
    nb = n // B
    assert nb * B == n

    wtp_flat = (W_tp.reshape(d, p * d)
                * (1.0 / (math.sqrt(d * p) * math.sqrt(avg_nodes))))
    wvsc_s = W_vsc * (1.0 / math.sqrt(d))
    wnsc_s = W_nsc * (1.0 / math.sqrt(d))
    wn2v_s = W_n2v * (1.0 / math.sqrt(d))
    batch3d = batch.reshape(nb, 1, B)
    batch_col = batch.reshape(n, 1)

    xvo, x_node_out = pl.pallas_call(
        _body,
        grid=(2, nb),
        in_specs=[
            pl.BlockSpec((B, d), lambda ph, i: (i, 0)),
            pl.BlockSpec((B, p), lambda ph, i: (i, 0)),
            pl.BlockSpec((1, 1, B), lambda ph, i: (i, 0, 0)),
            pl.BlockSpec((B, 1), lambda ph, i: (i, 0)),
            pl.BlockSpec((g, d), lambda ph, i: (0, 0)),
            pl.BlockSpec((d, d), lambda ph, i: (0, 0)),
            pl.BlockSpec((d, d), lambda ph, i: (0, 0)),
            pl.BlockSpec((d, p * d), lambda ph, i: (0, 0)),
            pl.BlockSpec((d, d), lambda ph, i: (0, 0)),
        ],
        out_specs=(
            pl.BlockSpec((g, d), lambda ph, i: (0, 0)),
            pl.BlockSpec((B, d), lambda ph, i: (ph * i, 0)),
        ),
        out_shape=(jax.ShapeDtypeStruct((g, d), jnp.float32),
                   jax.ShapeDtypeStruct((n, d), jnp.float32)),
        scratch_shapes=[
            pltpu.VMEM((g, d), jnp.float32),
            pltpu.VMEM((g, d), jnp.float32),
        ],
        compiler_params=pltpu.CompilerParams(
            dimension_semantics=("arbitrary", "arbitrary")),
    )(x_node, node_pos_sh, batch3d, batch_col, x_virtual,
      wvsc_s, wnsc_s, wtp_flat, wn2v_s)

    return (xvo, x_node_out)


# speedup vs baseline: 1.3411x; 1.3411x over previous
import math

import jax
import jax.numpy as jnp
from jax.experimental import pallas as pl
from jax.experimental.pallas import tpu as pltpu


def _stage_a_body(x_ref, pos_ref, batch_ref, wtp_ref, seg_ref):
    i = pl.program_id(0)

    @pl.when(i == 0)
    def _init():
        seg_ref[...] = jnp.zeros_like(seg_ref)

    x = x_ref[...]
    z = jnp.dot(x, wtp_ref[...], preferred_element_type=jnp.float32)
    pos = pos_ref[...]
    d = x.shape[1]
    p = pos.shape[1]
    n_over_g = pl.num_programs(0) * x.shape[0] / seg_ref.shape[0]
    scale = 1.0 / (math.sqrt(d * p) * math.sqrt(n_over_g))
    m = pos[:, 0:1] * z[:, 0:d]
    for j in range(1, p):
        m = m + pos[:, j:j + 1] * z[:, j * d:(j + 1) * d]
    m = m * scale
    bb = batch_ref[0]
    g = seg_ref.shape[0]
    onehot_t = (jax.lax.broadcasted_iota(jnp.int32, (g, bb.shape[1]), 0)
                == bb).astype(jnp.float32)
    seg_ref[...] += jnp.dot(onehot_t, m, preferred_element_type=jnp.float32)


def _stage_b_body(xv_ref, wvsc_ref, wn2v_ref, seg_ref, xvo_ref, y2_ref):
    d = xv_ref.shape[1]
    sv = jnp.dot(xv_ref[...], wvsc_ref[...],
                 preferred_element_type=jnp.float32) * (1.0 / math.sqrt(d))
    mv = seg_ref[...]
    mv = mv * jax.nn.sigmoid(mv)
    xvo = (sv + mv) * (1.0 / math.sqrt(2.0))
    xvo_ref[...] = xvo
    y2_ref[...] = jnp.dot(xvo, wn2v_ref[...],
                          preferred_element_type=jnp.float32) * (1.0 / math.sqrt(d))


def _stage_c_body(x_ref, batch_ref, wnsc_ref, y2_ref, out_ref):
    x = x_ref[...]
    d = x.shape[1]
    s = jnp.dot(x, wnsc_ref[...],
                preferred_element_type=jnp.float32) * (1.0 / math.sqrt(d))
    bb = batch_ref[0]
    g = y2_ref.shape[0]
    onehot_t = (jax.lax.broadcasted_iota(jnp.int32, (g, bb.shape[1]), 0)
                == bb).astype(jnp.float32)
    gath = jax.lax.dot_general(
        onehot_t, y2_ref[...], (((0,), (0,)), ((), ())),
        preferred_element_type=jnp.float32)
    out_ref[...] = (s + gath * jax.nn.sigmoid(gath)) * 0.5


def kernel(x_virtual, x_node, node_pos_sh, batch, W_vsc, W_nsc, W_tp, W_n2v):
    n, d = x_node.shape
    p = node_pos_sh.shape[1]
    g = x_virtual.shape[0]

    B = 2000
    nb = n // B
    assert nb * B == n

    wtp_flat = W_tp.reshape(d, p * d)
    batch3d = batch.reshape(nb, 1, B)

    seg = pl.pallas_call(
        _stage_a_body,
        grid=(nb,),
        in_specs=[
            pl.BlockSpec((B, d), lambda i: (i, 0)),
            pl.BlockSpec((B, p), lambda i: (i, 0)),
            pl.BlockSpec((1, 1, B), lambda i: (i, 0, 0)),
            pl.BlockSpec((d, p * d), lambda i: (0, 0)),
        ],
        out_specs=pl.BlockSpec((g, d), lambda i: (0, 0)),
        out_shape=jax.ShapeDtypeStruct((g, d), jnp.float32),
        compiler_params=pltpu.CompilerParams(
            dimension_semantics=("arbitrary",)),
    )(x_node, node_pos_sh, batch3d, wtp_flat)

    xvo, y2 = pl.pallas_call(
        _stage_b_body,
        out_shape=(jax.ShapeDtypeStruct((g, d), jnp.float32),
                   jax.ShapeDtypeStruct((g, d), jnp.float32)),
    )(x_virtual, W_vsc, W_n2v, seg)

    x_node_out = pl.pallas_call(
        _stage_c_body,
        grid=(nb,),
        in_specs=[
            pl.BlockSpec((B, d), lambda i: (i, 0)),
            pl.BlockSpec((1, 1, B), lambda i: (i, 0, 0)),
            pl.BlockSpec((d, d), lambda i: (0, 0)),
            pl.BlockSpec((g, d), lambda i: (0, 0)),
        ],
        out_specs=pl.BlockSpec((B, d), lambda i: (i, 0)),
        out_shape=jax.ShapeDtypeStruct((n, d), jnp.float32),
        compiler_params=pltpu.CompilerParams(
            dimension_semantics=("parallel",)),
    )(x_node, batch3d, W_nsc, y2)

    return (xvo, x_node_out)
